# baseline (speedup 1.0000x reference)
import functools

import jax
import jax.numpy as jnp
from jax import lax
from jax.experimental import pallas as pl
from jax.experimental.pallas import tpu as pltpu
from jax.experimental.pallas import tpu_sc as plsc

_B = 1
_S = 2048
_T = _B * _S
_D = 1024
_E = 8
_H = 2048
_K = 2
_NA = _T * _K
_BLK = 256
_NBLK = (_NA + _E * (_BLK - 1) + _BLK - 1) // _BLK
_NPAD = _NBLK * _BLK
_RTB = 256

_NW = 32
_DCH = 64
_CCH = 32


def _route_body(x_ref, gw_ref, rt_ref, gp_ref, be_ref, p0_ref, p1_ref,
                tpe_ref, aux_ref):
    x = x_ref[...]
    logits = lax.dot_general(x, gw_ref[...], (((1,), (1,)), ((), ())),
                             preferred_element_type=jnp.float32)
    mx = jnp.max(logits, axis=1, keepdims=True)
    ex = jnp.exp(logits - mx)
    probs = ex / jnp.sum(ex, axis=1, keepdims=True)

    iota = lax.broadcasted_iota(jnp.int32, (_T, _E), 1)
    v0 = jnp.max(probs, axis=1, keepdims=True)
    i0 = jnp.min(jnp.where(probs >= v0, iota, _E), axis=1, keepdims=True)
    m0 = iota == i0
    p2 = jnp.where(m0, -1.0, probs)
    v1 = jnp.max(p2, axis=1, keepdims=True)
    i1 = jnp.min(jnp.where(p2 >= v1, iota, _E), axis=1, keepdims=True)
    m1 = iota == i1
    m0f = m0.astype(jnp.float32)
    m1f = m1.astype(jnp.float32)

    denom = v0 + v1 + 1e-9
    g0 = v0 / denom
    g1 = v1 / denom

    c0_row = jnp.sum(m0f, axis=0, keepdims=True)
    c1_row = jnp.sum(m1f, axis=0, keepdims=True)
    cnt_row = c0_row + c1_row
    tpe_ref[...] = cnt_row
    m_mean = jnp.mean(probs, axis=0, keepdims=True)
    aux_ref[...] = jnp.reshape(_E * jnp.sum((cnt_row / _T) * m_mean), (1, 1))

    ri = lax.broadcasted_iota(jnp.int32, (_T, _T), 0)
    ci = lax.broadcasted_iota(jnp.int32, (_T, _T), 1)
    tri = (ci <= ri).astype(jnp.float32)
    mcat = jnp.concatenate([m0f, m1f], axis=1)
    ccat = lax.dot_general(tri, mcat, (((1,), (0,)), ((), ())),
                           preferred_element_type=jnp.float32)
    cum0 = ccat[:, :_E] - m0f
    cum1 = ccat[:, _E:] - m1f
    rank0 = jnp.sum(jnp.where(m0, cum0, 0.0), axis=1, keepdims=True)
    rank1 = jnp.sum(jnp.where(m1, cum1, 0.0), axis=1, keepdims=True)

    p_row = jnp.floor((cnt_row + (_BLK - 1)) * (1.0 / _BLK)) * _BLK
    ei = lax.broadcasted_iota(jnp.int32, (_E, _E), 0)
    ej = lax.broadcasted_iota(jnp.int32, (_E, _E), 1)
    tri_e = (ei <= ej).astype(jnp.float32)
    o_inc = lax.dot_general(p_row, tri_e, (((1,), (0,)), ((), ())),
                            preferred_element_type=jnp.float32)
    o_ex = o_inc - p_row

    sel0 = lambda row: jnp.sum(jnp.where(m0, row, 0.0), axis=1, keepdims=True)
    sel1 = lambda row: jnp.sum(jnp.where(m1, row, 0.0), axis=1, keepdims=True)
    pos0 = sel0(o_ex) + rank0
    pos1 = sel1(o_ex) + sel1(c0_row) + rank1
    p0_ref[...] = pos0.astype(jnp.int32)
    p1_ref[...] = pos1.astype(jnp.int32)

    bi = lax.broadcasted_iota(jnp.int32, (_NBLK, _E), 0).astype(jnp.float32)
    bei = lax.broadcasted_iota(jnp.int32, (_NBLK, _E), 1)
    base = bi * _BLK
    inb = jnp.logical_and(base >= o_ex, base < o_inc)
    be_ref[...] = jnp.sum(jnp.where(inb, bei, 0), axis=1, keepdims=True)

    tok = lax.broadcasted_iota(jnp.int32, (_T, 1), 0).astype(jnp.float32)
    for blk in range(_NPAD // _RTB):
        rr = (lax.broadcasted_iota(jnp.int32, (1, _RTB), 1)
              + blk * _RTB).astype(jnp.float32)
        sl = pl.ds(blk * _RTB, _RTB)
        hit0 = (pos0 == rr).astype(jnp.float32)
        hit1 = (pos1 == rr).astype(jnp.float32)
        dg = (((0,), (0,)), ((), ()))
        rtb = (lax.dot_general(tok, hit0, dg, preferred_element_type=jnp.float32)
               + lax.dot_general(tok, hit1, dg,
                                 preferred_element_type=jnp.float32))
        gpb = (lax.dot_general(g0, hit0, dg, preferred_element_type=jnp.float32)
               + lax.dot_general(g1, hit1, dg,
                                 preferred_element_type=jnp.float32))
        rt_ref[0, sl] = rtb[0].astype(jnp.int32)
        gp_ref[0, sl] = gpb[0]


def _route(xt, gate_W):
    return pl.pallas_call(
        _route_body,
        in_specs=[
            pl.BlockSpec((_T, _D), lambda: (0, 0)),
            pl.BlockSpec((_E, _D), lambda: (0, 0)),
        ],
        out_specs=[
            pl.BlockSpec((1, _NPAD), lambda: (0, 0)),
            pl.BlockSpec((1, _NPAD), lambda: (0, 0)),
            pl.BlockSpec((_NBLK, 1), lambda: (0, 0)),
            pl.BlockSpec((_T, 1), lambda: (0, 0)),
            pl.BlockSpec((_T, 1), lambda: (0, 0)),
            pl.BlockSpec((1, _E), lambda: (0, 0)),
            pl.BlockSpec((1, 1), lambda: (0, 0)),
        ],
        out_shape=[
            jax.ShapeDtypeStruct((1, _NPAD), jnp.int32),
            jax.ShapeDtypeStruct((1, _NPAD), jnp.float32),
            jax.ShapeDtypeStruct((_NBLK, 1), jnp.int32),
            jax.ShapeDtypeStruct((_T, 1), jnp.int32),
            jax.ShapeDtypeStruct((_T, 1), jnp.int32),
            jax.ShapeDtypeStruct((1, _E), jnp.float32),
            jax.ShapeDtypeStruct((1, 1), jnp.float32),
        ],
    )(xt, gate_W)


@functools.cache
def _get_dispatch():
    mesh = plsc.VectorSubcoreMesh(core_axis_name="c", subcore_axis_name="s")

    @functools.partial(
        pl.kernel, mesh=mesh,
        out_type=jax.ShapeDtypeStruct((_NPAD, _D), jnp.float32),
        scratch_types=[
            pltpu.VMEM((_DCH,), jnp.int32),
            pltpu.VMEM((_DCH, _D), jnp.float32),
            pltpu.SemaphoreType.DMA,
        ],
    )
    def _dispatch(rt_hbm, x_hbm, xs_hbm, idx_v, rows_v, sem):
        wid = lax.axis_index("s") * 2 + lax.axis_index("c")
        per_w = _NPAD // _NW
        base_w = wid * per_w
        for j in range(per_w // _DCH):
            base = base_w + j * _DCH
            pltpu.sync_copy(rt_hbm.at[pl.ds(base, _DCH)], idx_v)
            pltpu.async_copy(x_hbm.at[idx_v], rows_v, sem).wait()
            pltpu.sync_copy(rows_v, xs_hbm.at[pl.ds(base, _DCH)])

    return _dispatch


def _ffn_body(be_ref, xs_ref, w1_ref, b1_ref, w2_ref, b2_ref, g_ref, out_ref):
    x = xs_ref[...].astype(jnp.bfloat16)
    h = lax.dot_general(x, w1_ref[0].astype(jnp.bfloat16),
                        (((1,), (0,)), ((), ())),
                        preferred_element_type=jnp.float32) + b1_ref[0]
    h = jnp.maximum(h, 0.0).astype(jnp.bfloat16)
    o = lax.dot_general(h, w2_ref[0].astype(jnp.bfloat16),
                        (((1,), (0,)), ((), ())),
                        preferred_element_type=jnp.float32) + b2_ref[0]
    out_ref[...] = g_ref[...] * o


def _ffn(be, xs, W1, b1r, W2, b2r, gates):
    grid_spec = pltpu.PrefetchScalarGridSpec(
        num_scalar_prefetch=1,
        grid=(_NBLK,),
        in_specs=[
            pl.BlockSpec((_BLK, _D), lambda i, be: (i, 0)),
            pl.BlockSpec((1, _D, _H), lambda i, be: (be[i], 0, 0)),
            pl.BlockSpec((1, 1, _H), lambda i, be: (be[i], 0, 0)),
            pl.BlockSpec((1, _H, _D), lambda i, be: (be[i], 0, 0)),
            pl.BlockSpec((1, 1, _D), lambda i, be: (be[i], 0, 0)),
            pl.BlockSpec((_BLK, 1), lambda i, be: (i, 0)),
        ],
        out_specs=pl.BlockSpec((_BLK, _D), lambda i, be: (i, 0)),
    )
    return pl.pallas_call(
        _ffn_body,
        grid_spec=grid_spec,
        out_shape=jax.ShapeDtypeStruct((_NPAD, _D), jnp.float32),
        compiler_params=pltpu.CompilerParams(
            dimension_semantics=("arbitrary",)),
    )(be, xs, W1, b1r, W2, b2r, gates)


@functools.cache
def _get_combine():
    mesh = plsc.VectorSubcoreMesh(core_axis_name="c", subcore_axis_name="s")

    @functools.partial(
        pl.kernel, mesh=mesh,
        out_type=jax.ShapeDtypeStruct((_T, _D), jnp.float32),
        scratch_types=[
            pltpu.VMEM((_CCH,), jnp.int32),
            pltpu.VMEM((_CCH,), jnp.int32),
            pltpu.VMEM((_CCH, _D), jnp.float32),
            pltpu.VMEM((_CCH, _D), jnp.float32),
            pltpu.SemaphoreType.DMA,
        ],
    )
    def _combine(pos0_hbm, pos1_hbm, rows_hbm, y_hbm, i0_v, i1_v, a_v, b_v,
                 sem):
        wid = lax.axis_index("s") * 2 + lax.axis_index("c")
        per_w = _T // _NW
        base_w = wid * per_w
        for j in range(per_w // _CCH):
            base = base_w + j * _CCH
            pltpu.sync_copy(pos0_hbm.at[pl.ds(base, _CCH)], i0_v)
            pltpu.sync_copy(pos1_hbm.at[pl.ds(base, _CCH)], i1_v)
            pltpu.async_copy(rows_hbm.at[i0_v], a_v, sem).wait()
            pltpu.async_copy(rows_hbm.at[i1_v], b_v, sem).wait()

            def row_add(r, _):
                for c in range(_D // 16):
                    sl = pl.ds(c * 16, 16)
                    a_v[r, sl] = a_v[r, sl] + b_v[r, sl]
                return 0

            lax.fori_loop(0, _CCH, row_add, 0)
            pltpu.sync_copy(a_v, y_hbm.at[pl.ds(base, _CCH)])

    return _combine


def kernel(x, gate_W, W1, b1, W2, b2):
    xt = x.reshape(_T, _D)
    rt2, gp2, be2, p0, p1, tpe, aux = _route(xt, gate_W)
    rt = rt2.reshape(_NPAD)
    gates = gp2.reshape(_NPAD, 1)
    be = be2.reshape(_NBLK)

    xs = _get_dispatch()(rt, xt)
    out_rows = _ffn(be, xs, W1, b1.reshape(_E, 1, _H), W2,
                    b2.reshape(_E, 1, _D), gates)
    y = _get_combine()(p0.reshape(_T), p1.reshape(_T), out_rows)
    return (y.reshape(_B, _S, _D), aux[0, 0], tpe[0])


# speedup vs baseline: 1.6240x; 1.6240x over previous
import jax
import jax.numpy as jnp
from jax.experimental import pallas as pl
from jax.experimental.pallas import tpu as pltpu

_B = 1
_S = 2048
_T = _B * _S
_D = 1024
_E = 8
_H = 2048
_HB = 2
_HBS = _H // _HB


def _moe_body(x_ref, gw_ref, w1_ref, b1_ref, w2_ref, b2_ref,
              y_ref, tpe_ref, aux_ref, comb_ref):
    e = pl.program_id(0)
    hb = pl.program_id(1)

    @pl.when(jnp.logical_and(e == 0, hb == 0))
    def _route():
        x = x_ref[...]
        logits = jax.lax.dot_general(
            x, gw_ref[...], (((1,), (1,)), ((), ())),
            preferred_element_type=jnp.float32)
        mx = jnp.max(logits, axis=1, keepdims=True)
        ex = jnp.exp(logits - mx)
        probs = ex / jnp.sum(ex, axis=1, keepdims=True)

        iota = jax.lax.broadcasted_iota(jnp.int32, (_T, _E), 1)
        v0 = jnp.max(probs, axis=1, keepdims=True)
        i0 = jnp.min(jnp.where(probs >= v0, iota, _E), axis=1, keepdims=True)
        m0 = iota == i0
        p2 = jnp.where(m0, -1.0, probs)
        v1 = jnp.max(p2, axis=1, keepdims=True)
        i1 = jnp.min(jnp.where(p2 >= v1, iota, _E), axis=1, keepdims=True)
        m1 = iota == i1

        denom = v0 + v1 + 1e-9
        comb = jnp.where(m0, v0 / denom, 0.0) + jnp.where(m1, v1 / denom, 0.0)
        comb_ref[...] = comb

        counts = jnp.sum((m0 | m1).astype(jnp.float32), axis=0, keepdims=True)
        tpe_ref[...] = counts
        m_mean = jnp.mean(probs, axis=0, keepdims=True)
        aux = _E * jnp.sum((counts / _T) * m_mean)
        aux_ref[...] = jnp.reshape(aux, (1, 1))

    iota = jax.lax.broadcasted_iota(jnp.int32, (_T, _E), 1)
    w = jnp.sum(jnp.where(iota == e, comb_ref[...], 0.0), axis=1,
                keepdims=True)

    x = x_ref[...].astype(jnp.bfloat16)
    h = jax.lax.dot_general(
        x, w1_ref[0].astype(jnp.bfloat16), (((1,), (0,)), ((), ())),
        preferred_element_type=jnp.float32) + b1_ref[0]
    h = (w * jnp.maximum(h, 0.0)).astype(jnp.bfloat16)
    contrib = jax.lax.dot_general(
        h, w2_ref[0].astype(jnp.bfloat16), (((1,), (0,)), ((), ())),
        preferred_element_type=jnp.float32)
    contrib = jnp.where(hb == 0, contrib + w * b2_ref[0], contrib)

    @pl.when(jnp.logical_and(e == 0, hb == 0))
    def _init():
        y_ref[...] = contrib

    @pl.when(jnp.logical_or(e > 0, hb > 0))
    def _acc():
        y_ref[...] = y_ref[...] + contrib


def kernel(x, gate_W, W1, b1, W2, b2):
    xt = x.reshape(_T, _D)
    y, tpe, aux = pl.pallas_call(
        _moe_body,
        grid=(_E, _HB),
        in_specs=[
            pl.BlockSpec((_T, _D), lambda e, h: (0, 0)),
            pl.BlockSpec((_E, _D), lambda e, h: (0, 0)),
            pl.BlockSpec((1, _D, _HBS), lambda e, h: (e, 0, h)),
            pl.BlockSpec((1, 1, _HBS), lambda e, h: (e * _HB + h, 0, 0)),
            pl.BlockSpec((1, _HBS, _D), lambda e, h: (e, h, 0)),
            pl.BlockSpec((1, 1, _D), lambda e, h: (e, 0, 0)),
        ],
        out_specs=[
            pl.BlockSpec((_T, _D), lambda e, h: (0, 0)),
            pl.BlockSpec((1, _E), lambda e, h: (0, 0)),
            pl.BlockSpec((1, 1), lambda e, h: (0, 0)),
        ],
        out_shape=[
            jax.ShapeDtypeStruct((_T, _D), jnp.float32),
            jax.ShapeDtypeStruct((1, _E), jnp.float32),
            jax.ShapeDtypeStruct((1, 1), jnp.float32),
        ],
        scratch_shapes=[pltpu.VMEM((_T, _E), jnp.float32)],
        compiler_params=pltpu.CompilerParams(
            dimension_semantics=("arbitrary", "arbitrary")),
    )(xt, gate_W, W1, b1.reshape(_E * _HB, 1, _HBS), W2,
      b2.reshape(_E, 1, _D))
    return (y.reshape(_B, _S, _D), aux[0, 0], tpe[0])
